# Initial kernel scaffold; baseline (speedup 1.0000x reference)
import jax
import jax.numpy as jnp
from jax.experimental import pallas as pl


def kernel(x, edge_attr, edge_index, W1, b1, W2, b2):
    raise NotImplementedError("write your pallas kernel here")


# speedup vs baseline: 1.9974x; 1.9974x over previous
import functools

import jax
import jax.numpy as jnp
from jax import lax
from jax.experimental import pallas as pl
from jax.experimental.pallas import tpu as pltpu
from jax.experimental.pallas import tpu_sc as plsc

N = 10000
E = 320000
D = 128
DE = 16
H = 128

NC = 2
NS = 16
NW = NC * NS
EB = 40
E_PER_TILE = E // NW
NBLK = E_PER_TILE // EB


def _pq_body(x_ref, wa_ref, wb_ref, p_ref, q_ref):
    xb = x_ref[...]
    p_ref[...] = jnp.dot(xb, wa_ref[...], preferred_element_type=jnp.float32,
                         precision=jax.lax.Precision.HIGHEST)
    q_ref[...] = jnp.dot(xb, wb_ref[...], preferred_element_type=jnp.float32,
                         precision=jax.lax.Precision.HIGHEST)


def _compute_pq(x, wa, wb):
    blk = 1000
    return pl.pallas_call(
        _pq_body,
        grid=(N // blk,),
        in_specs=[
            pl.BlockSpec((blk, D), lambda i: (i, 0)),
            pl.BlockSpec((D, H), lambda i: (0, 0)),
            pl.BlockSpec((D, H), lambda i: (0, 0)),
        ],
        out_specs=[
            pl.BlockSpec((blk, H), lambda i: (i, 0)),
            pl.BlockSpec((blk, H), lambda i: (i, 0)),
        ],
        out_shape=[
            jax.ShapeDtypeStruct((N, H), jnp.float32),
            jax.ShapeDtypeStruct((N, H), jnp.float32),
        ],
    )(x, wa, wb)


def _r_body(ea_ref, wc_ref, b1_ref, r_ref):
    r_ref[...] = (
        jnp.dot(ea_ref[...], wc_ref[...], preferred_element_type=jnp.float32,
                precision=jax.lax.Precision.HIGHEST)
        + b1_ref[...]
    )


def _compute_r(edge_attr, wc, b1r):
    blk = 8000
    return pl.pallas_call(
        _r_body,
        grid=(E // blk,),
        in_specs=[
            pl.BlockSpec((blk, DE), lambda i: (i, 0)),
            pl.BlockSpec((DE, H), lambda i: (0, 0)),
            pl.BlockSpec((1, H), lambda i: (0, 0)),
        ],
        out_specs=pl.BlockSpec((blk, H), lambda i: (i, 0)),
        out_shape=jax.ShapeDtypeStruct((E, H), jnp.float32),
    )(edge_attr, wc, b1r)


def _edge_body(p_hbm, q_hbm, r_hbm, src_hbm, dst_hbm, w2_hbm, zero_hbm,
               out_hbm, dsti_v, srci_v, pd_v, qs_v, r_v, s_v, w2_v, shared,
               sem_p, sem_q, sem_r):
    c = lax.axis_index("c")
    s = lax.axis_index("s")
    wid = c * NS + s

    @pl.when(s == 0)
    def _():
        pltpu.sync_copy(zero_hbm, shared)

    pltpu.sync_copy(w2_hbm, w2_v)

    zpad = jnp.zeros((16,), jnp.float32)
    for e in range(EB):
        for g in range(1, H // 16):
            s_v[e, pl.ds(g * 16, 16)] = zpad

    plsc.subcore_barrier()

    w2g = [w2_v[pl.ds(g * 16, 16)] for g in range(H // 16)]
    tile_base = wid * E_PER_TILE

    def block(b, carry):
        off = tile_base + b * EB
        pltpu.sync_copy(dst_hbm.at[pl.ds(off, EB)], dsti_v)
        pltpu.sync_copy(src_hbm.at[pl.ds(off, EB)], srci_v)
        cp_p = pltpu.async_copy(p_hbm.at[dsti_v], pd_v, sem_p)
        cp_q = pltpu.async_copy(q_hbm.at[srci_v], qs_v, sem_q)
        cp_r = pltpu.async_copy(r_hbm.at[pl.ds(off, EB)], r_v, sem_r)
        cp_p.wait()
        cp_q.wait()
        cp_r.wait()
        for e in range(EB):
            acc = jnp.zeros((16,), jnp.float32)
            for g in range(H // 16):
                t = (pd_v[e, pl.ds(g * 16, 16)]
                     + qs_v[e, pl.ds(g * 16, 16)]
                     + r_v[e, pl.ds(g * 16, 16)])
                acc = acc + jnp.maximum(t, 0.0) * w2g[g]
            s_v[e, pl.ds(0, 16)] = acc
        pltpu.sync_copy(s_v, shared.at[dsti_v], add=True)
        return carry

    lax.fori_loop(0, NBLK, block, 0)

    plsc.subcore_barrier()

    @pl.when(s == 0)
    def _():
        pltpu.sync_copy(shared, out_hbm.at[c])


@functools.lru_cache(maxsize=None)
def _make_edge_kernel():
    return functools.partial(
        pl.kernel,
        out_type=jax.ShapeDtypeStruct((NC, N, H), jnp.float32),
        mesh=plsc.VectorSubcoreMesh(
            core_axis_name="c", subcore_axis_name="s", num_cores=NC,
            num_subcores=NS),
        scratch_types=[
            pltpu.VMEM((EB,), jnp.int32),
            pltpu.VMEM((EB,), jnp.int32),
            pltpu.VMEM((EB, H), jnp.float32),
            pltpu.VMEM((EB, H), jnp.float32),
            pltpu.VMEM((EB, H), jnp.float32),
            pltpu.VMEM((EB, H), jnp.float32),
            pltpu.VMEM((H,), jnp.float32),
            pltpu.VMEM_SHARED((N, H), jnp.float32),
            pltpu.SemaphoreType.DMA,
            pltpu.SemaphoreType.DMA,
            pltpu.SemaphoreType.DMA,
        ],
    )(lambda p, q, r, src, dst, w2, zero, out, *scratch:
          _edge_body(p, q, r, src, dst, w2, zero, out, *scratch))


def _fin_body(part_ref, b2_ref, h_ref):
    h_ref[...] = jnp.sum(part_ref[...], axis=(0, 2))[None, :] + b2_ref[...]


def _combine(partials, b2r):
    return pl.pallas_call(
        _fin_body,
        in_specs=[
            pl.BlockSpec((NC, N, H), lambda: (0, 0, 0)),
            pl.BlockSpec((1, 1), lambda: (0, 0)),
        ],
        out_specs=pl.BlockSpec((1, N), lambda: (0, 0)),
        out_shape=jax.ShapeDtypeStruct((1, N), jnp.float32),
    )(partials, b2r)


def kernel(x, edge_attr, edge_index, W1, b1, W2, b2):
    wa = W1[:D]
    wb = W1[D:2 * D]
    wc = W1[2 * D:]
    p, q = _compute_pq(x, wa, wb)
    r = _compute_r(edge_attr, wc, b1.reshape(1, H))
    src = edge_index[0]
    dst = edge_index[1]
    zero = jnp.zeros((N, H), jnp.float32)
    partials = _make_edge_kernel()(p, q, r, src, dst, W2.reshape(H), zero)
    h = _combine(partials, b2.reshape(1, 1))
    return h.reshape(N, 1)


# speedup vs baseline: 3.0514x; 1.5277x over previous
import functools

import jax
import jax.numpy as jnp
from jax import lax
from jax.experimental import pallas as pl
from jax.experimental.pallas import tpu as pltpu
from jax.experimental.pallas import tpu_sc as plsc

N = 10000
E = 320000
D = 128
DE = 16
H = 128
NG = H // 16

NC = 2
NS = 16
NW = NC * NS
EB = 40
E_PER_TILE = E // NW
NBLK = E_PER_TILE // EB
N_PAD = 10240


def _pq_body(x_ref, wa_ref, wb_ref, p_ref, q_ref):
    xb = x_ref[...]
    p_ref[...] = jnp.dot(xb, wa_ref[...], preferred_element_type=jnp.float32,
                         precision=jax.lax.Precision.HIGHEST)
    q_ref[...] = jnp.dot(xb, wb_ref[...], preferred_element_type=jnp.float32,
                         precision=jax.lax.Precision.HIGHEST)


def _compute_pq(x, wa, wb):
    blk = 1000
    return pl.pallas_call(
        _pq_body,
        grid=(N // blk,),
        in_specs=[
            pl.BlockSpec((blk, D), lambda i: (i, 0)),
            pl.BlockSpec((D, H), lambda i: (0, 0)),
            pl.BlockSpec((D, H), lambda i: (0, 0)),
        ],
        out_specs=[
            pl.BlockSpec((blk, H), lambda i: (i, 0)),
            pl.BlockSpec((blk, H), lambda i: (i, 0)),
        ],
        out_shape=[
            jax.ShapeDtypeStruct((N, H), jnp.float32),
            jax.ShapeDtypeStruct((N, H), jnp.float32),
        ],
    )(x, wa, wb)


def _r_body(ea_ref, wc_ref, b1_ref, r_ref):
    r_ref[...] = (
        jnp.dot(ea_ref[...], wc_ref[...], preferred_element_type=jnp.float32,
                precision=jax.lax.Precision.HIGHEST)
        + b1_ref[...]
    )


def _compute_r(edge_attr, wc, b1r):
    blk = 8000
    return pl.pallas_call(
        _r_body,
        grid=(E // blk,),
        in_specs=[
            pl.BlockSpec((blk, DE), lambda i: (i, 0)),
            pl.BlockSpec((DE, H), lambda i: (0, 0)),
            pl.BlockSpec((1, H), lambda i: (0, 0)),
        ],
        out_specs=pl.BlockSpec((blk, H), lambda i: (i, 0)),
        out_shape=jax.ShapeDtypeStruct((E, H), jnp.float32),
    )(edge_attr, wc, b1r)


def _edge_body(p_hbm, q_hbm, r_hbm, src1_hbm, dst1_hbm, w2_hbm, zero_hbm,
               out_hbm, dsti_v, srci_v, pd_v, qs_v, r_v, s_v, dstb_v, w2_v,
               shared, sems):
    c = lax.axis_index("c")
    s = lax.axis_index("s")
    wid = c * NS + s

    @pl.when(s == 0)
    def _():
        pltpu.sync_copy(zero_hbm, shared)

    ebase = pl.multiple_of(wid * E_PER_TILE, 8)
    pltpu.sync_copy(dst1_hbm.at[pl.ds(ebase, E_PER_TILE)], dsti_v)
    pltpu.sync_copy(src1_hbm.at[pl.ds(ebase, E_PER_TILE)], srci_v)
    pltpu.sync_copy(w2_hbm, w2_v)
    plsc.subcore_barrier()

    zero16 = jnp.zeros((16,), jnp.float32)
    w2g = [w2_v[pl.ds(g * 16, 16)] for g in range(NG)]
    lanes = lax.iota(jnp.int32, 16)
    perms = [lanes ^ sh for sh in (8, 4, 2, 1)]
    shift8 = jnp.minimum(lanes + 8, 15)
    lane_lt8 = lanes < 8
    dummy_idx = jnp.full((16,), N, jnp.int32)
    row_base = wid * NBLK

    sem_p = [sems.at[0], sems.at[1]]
    sem_q = [sems.at[2], sems.at[3]]
    sem_r = [sems.at[4], sems.at[5]]
    sem_s = [sems.at[6], sems.at[7]]

    def _boff(b):
        return pl.multiple_of(b * EB, 8)

    def _roff(b):
        return pl.multiple_of((row_base + b) * EB, 8)

    def start(b, k):
        pltpu.async_copy(
            p_hbm.at[dsti_v.at[pl.ds(_boff(b), EB)]], pd_v.at[k], sem_p[k])
        pltpu.async_copy(
            q_hbm.at[srci_v.at[pl.ds(_boff(b), EB)]], qs_v.at[k], sem_q[k])
        pltpu.async_copy(r_hbm.at[pl.ds(_roff(b), EB)], r_v.at[k], sem_r[k])

    def wait(b, k):
        pltpu.make_async_copy(
            p_hbm.at[dsti_v.at[pl.ds(_boff(b), EB)]], pd_v.at[k],
            sem_p[k]).wait()
        pltpu.make_async_copy(
            q_hbm.at[srci_v.at[pl.ds(_boff(b), EB)]], qs_v.at[k],
            sem_q[k]).wait()
        pltpu.make_async_copy(
            r_hbm.at[pl.ds(_roff(b), EB)], r_v.at[k], sem_r[k]).wait()

    def scatter(k):
        pltpu.async_copy(s_v.at[k], shared.at[dstb_v.at[k]], sem_s[k],
                         add=True)

    def wait_scatter(k):
        pltpu.make_async_copy(
            s_v.at[k], shared.at[dstb_v.at[k]], sem_s[k]).wait()

    def compute(b, k):
        boff = _boff(b)
        dstb_v[k, pl.ds(0, 16)] = dsti_v[pl.ds(boff, 16)]
        dstb_v[k, pl.ds(16, 16)] = dsti_v[pl.ds(boff + 16, 16)]
        tail_raw = dsti_v[pl.ds(boff + 24, 16)]
        tail_idx = jnp.where(lane_lt8, tail_raw[shift8], dummy_idx)
        dstb_v[k, pl.ds(32, 16)] = tail_idx

        svec = zero16
        for e in range(EB):
            acc = zero16
            for g in range(NG):
                t = (pd_v[k, e, pl.ds(g * 16, 16)]
                     + qs_v[k, e, pl.ds(g * 16, 16)]
                     + r_v[k, e, pl.ds(g * 16, 16)])
                acc = acc + jnp.maximum(t, 0.0) * w2g[g]
            for prm in perms:
                acc = acc + acc[prm]
            lane = e % 16 if e < 32 else e - 32
            svec = jnp.where(lanes == lane, acc, svec)
            if e in (15, 31):
                s_v[k, pl.ds(e - 15, 16)] = svec
                svec = zero16
            elif e == EB - 1:
                svec = jnp.where(lane_lt8, svec, zero16)
                s_v[k, pl.ds(32, 16)] = svec

    start(0, 0)

    def pair(j, carry):
        i = j * 2
        start(i + 1, 1)
        wait(i, 0)

        @pl.when(j > 0)
        def _():
            wait_scatter(0)

        compute(i, 0)
        scatter(0)

        @pl.when(i + 2 < NBLK)
        def _():
            start(i + 2, 0)

        wait(i + 1, 1)

        @pl.when(j > 0)
        def _():
            wait_scatter(1)

        compute(i + 1, 1)
        scatter(1)
        return carry

    lax.fori_loop(0, NBLK // 2, pair, 0)
    wait_scatter(0)
    wait_scatter(1)

    plsc.subcore_barrier()

    @pl.when(s == 0)
    def _():
        pltpu.sync_copy(shared, out_hbm.at[c])


@functools.lru_cache(maxsize=None)
def _make_edge_kernel():
    return functools.partial(
        pl.kernel,
        out_type=jax.ShapeDtypeStruct((NC, N_PAD), jnp.float32),
        mesh=plsc.VectorSubcoreMesh(
            core_axis_name="c", subcore_axis_name="s", num_cores=NC,
            num_subcores=NS),
        scratch_types=[
            pltpu.VMEM((E_PER_TILE,), jnp.int32),
            pltpu.VMEM((E_PER_TILE,), jnp.int32),
            pltpu.VMEM((2, EB, H), jnp.float32),
            pltpu.VMEM((2, EB, H), jnp.float32),
            pltpu.VMEM((2, EB, H), jnp.float32),
            pltpu.VMEM((2, 48), jnp.float32),
            pltpu.VMEM((2, 48), jnp.int32),
            pltpu.VMEM((H,), jnp.float32),
            pltpu.VMEM_SHARED((N_PAD,), jnp.float32),
            pltpu.SemaphoreType.DMA((8,)),
        ],
    )(lambda p, q, r, src1, dst1, w2, zero, out, *scratch:
          _edge_body(p, q, r, src1, dst1, w2, zero, out, *scratch))


def _fin_body(part_ref, b2_ref, h_ref):
    h_ref[...] = jnp.sum(part_ref[...], axis=0, keepdims=True) + b2_ref[...]


def _combine(partials, b2r):
    return pl.pallas_call(
        _fin_body,
        in_specs=[
            pl.BlockSpec((NC, N_PAD), lambda: (0, 0)),
            pl.BlockSpec((1, 1), lambda: (0, 0)),
        ],
        out_specs=pl.BlockSpec((1, N_PAD), lambda: (0, 0)),
        out_shape=jax.ShapeDtypeStruct((1, N_PAD), jnp.float32),
    )(partials, b2r)


def kernel(x, edge_attr, edge_index, W1, b1, W2, b2):
    wa = W1[:D]
    wb = W1[D:2 * D]
    wc = W1[2 * D:]
    p, q = _compute_pq(x, wa, wb)
    r = _compute_r(edge_attr, wc, b1.reshape(1, H))
    src1 = edge_index[0]
    dst1 = edge_index[1]
    zero = jnp.zeros((N_PAD,), jnp.float32)
    partials = _make_edge_kernel()(p, q, r, src1, dst1, W2.reshape(H), zero)
    h = _combine(partials, b2.reshape(1, 1))
    return h[0, :N].reshape(N, 1)
